# baseline (speedup 1.0000x reference)
import functools

import jax
import jax.numpy as jnp
from jax.experimental import pallas as pl
from jax.experimental.pallas import tpu as pltpu


def _lstm_body(x_ref, lens_ref, wih_ref, whh_ref, b_ref,
               out_ref, hn_ref, cn_ref,
               g_s, h_s, c_s, *, tc, b, h):
    i = pl.program_id(0)
    nblk = pl.num_programs(0)

    @pl.when(i == 0)
    def _init():
        h_s[...] = jnp.zeros_like(h_s)
        c_s[...] = jnp.zeros_like(c_s)

    xm = x_ref[...].reshape(b * tc, x_ref.shape[-1])
    g_s[...] = (jnp.dot(xm, wih_ref[...], preferred_element_type=jnp.float32)
                + b_ref[...]).reshape(b, tc, 4 * h)

    lens_v = lens_ref[...]

    def step(t, carry):
        hc, cc = carry
        hb = hc.astype(jnp.bfloat16)
        gt = g_s[:, t, :]
        ig = jax.nn.sigmoid(gt[:, :h]
                            + jnp.dot(hb, whh_ref[:, :h],
                                      preferred_element_type=jnp.float32))
        gg = jnp.tanh(gt[:, 2 * h:3 * h]
                      + jnp.dot(hb, whh_ref[:, 2 * h:3 * h],
                                preferred_element_type=jnp.float32))
        fg = jax.nn.sigmoid(gt[:, h:2 * h]
                            + jnp.dot(hb, whh_ref[:, h:2 * h],
                                      preferred_element_type=jnp.float32))
        og = jax.nn.sigmoid(gt[:, 3 * h:]
                            + jnp.dot(hb, whh_ref[:, 3 * h:],
                                      preferred_element_type=jnp.float32))
        c_new = fg * cc + ig * gg
        h_new = og * jnp.tanh(c_new)
        tg = i * tc + t
        m = (tg < lens_v).astype(jnp.float32)
        out_ref[:, t, :] = m * h_new
        return (hc + m * (h_new - hc), cc + m * (c_new - cc))

    hf, cf = jax.lax.fori_loop(0, tc, step, (h_s[...], c_s[...]), unroll=2)
    h_s[...] = hf
    c_s[...] = cf

    @pl.when(i == nblk - 1)
    def _fin():
        hn_ref[...] = hf[None]
        cn_ref[...] = cf[None]


def kernel(input, seq_lengths, W_ih, W_hh, b_ih, b_hh):
    B, T, D = input.shape
    H = W_hh.shape[1]
    H4 = 4 * H
    TC = 128

    wih_t = jnp.transpose(W_ih)
    whh_t = jnp.transpose(W_hh).astype(jnp.bfloat16)
    bias = (b_ih + b_hh)[None, :]
    lens = jnp.broadcast_to(
        seq_lengths.astype(jnp.int32)[:, None], (B, H))

    body = functools.partial(_lstm_body, tc=TC, b=B, h=H)
    out_tm, h_n, c_n = pl.pallas_call(
        body,
        grid=(T // TC,),
        in_specs=[
            pl.BlockSpec((B, TC, D), lambda i: (0, i, 0)),
            pl.BlockSpec((B, H), lambda i: (0, 0)),
            pl.BlockSpec((D, H4), lambda i: (0, 0)),
            pl.BlockSpec((H, H4), lambda i: (0, 0)),
            pl.BlockSpec((1, H4), lambda i: (0, 0)),
        ],
        out_specs=[
            pl.BlockSpec((B, TC, H), lambda i: (0, i, 0)),
            pl.BlockSpec((1, B, H), lambda i: (0, 0, 0)),
            pl.BlockSpec((1, B, H), lambda i: (0, 0, 0)),
        ],
        out_shape=[
            jax.ShapeDtypeStruct((B, T, H), jnp.float32),
            jax.ShapeDtypeStruct((1, B, H), jnp.float32),
            jax.ShapeDtypeStruct((1, B, H), jnp.float32),
        ],
        scratch_shapes=[
            pltpu.VMEM((B, TC, H4), jnp.float32),
            pltpu.VMEM((B, H), jnp.float32),
            pltpu.VMEM((B, H), jnp.float32),
        ],
    )(input, lens, wih_t, whh_t, bias)

    return (out_tm, h_n, c_n)


# speedup vs baseline: 1.0133x; 1.0133x over previous
import functools

import jax
import jax.numpy as jnp
from jax.experimental import pallas as pl
from jax.experimental.pallas import tpu as pltpu


def _lstm_body(x_ref, lens_ref, wih_ref, whh_ref, b_ref,
               out_ref, hn_ref, cn_ref,
               g_s, h_s, c_s, *, tc, b, h):
    i = pl.program_id(0)
    nblk = pl.num_programs(0)

    @pl.when(i == 0)
    def _init():
        h_s[...] = jnp.zeros_like(h_s)
        c_s[...] = jnp.zeros_like(c_s)

    xm = x_ref[...].reshape(b * tc, x_ref.shape[-1])
    g_s[...] = (jnp.dot(xm, wih_ref[...], preferred_element_type=jnp.float32)
                + b_ref[...]).reshape(b, tc, 4 * h)

    lens_v = lens_ref[...]

    def step(t, carry):
        hc, cc = carry
        hb = hc.astype(jnp.bfloat16)
        gt = g_s[:, t, :]
        ig = jax.nn.sigmoid(gt[:, :h]
                            + jnp.dot(hb, whh_ref[:, :h],
                                      preferred_element_type=jnp.float32))
        gg = jnp.tanh(gt[:, 2 * h:3 * h]
                      + jnp.dot(hb, whh_ref[:, 2 * h:3 * h],
                                preferred_element_type=jnp.float32))
        fg = jax.nn.sigmoid(gt[:, h:2 * h]
                            + jnp.dot(hb, whh_ref[:, h:2 * h],
                                      preferred_element_type=jnp.float32))
        og = jax.nn.sigmoid(gt[:, 3 * h:]
                            + jnp.dot(hb, whh_ref[:, 3 * h:],
                                      preferred_element_type=jnp.float32))
        c_new = fg * cc + ig * gg
        h_new = og * jnp.tanh(c_new)
        tg = i * tc + t
        m = (tg < lens_v).astype(jnp.float32)
        out_ref[:, t, :] = m * h_new
        return (hc + m * (h_new - hc), cc + m * (c_new - cc))

    hf, cf = jax.lax.fori_loop(0, tc, step, (h_s[...], c_s[...]), unroll=8)
    h_s[...] = hf
    c_s[...] = cf

    @pl.when(i == nblk - 1)
    def _fin():
        hn_ref[...] = hf[None]
        cn_ref[...] = cf[None]


def kernel(input, seq_lengths, W_ih, W_hh, b_ih, b_hh):
    B, T, D = input.shape
    H = W_hh.shape[1]
    H4 = 4 * H
    TC = 128

    wih_t = jnp.transpose(W_ih)
    whh_t = jnp.transpose(W_hh).astype(jnp.bfloat16)
    bias = (b_ih + b_hh)[None, :]
    lens = jnp.broadcast_to(
        seq_lengths.astype(jnp.int32)[:, None], (B, H))

    body = functools.partial(_lstm_body, tc=TC, b=B, h=H)
    out_tm, h_n, c_n = pl.pallas_call(
        body,
        grid=(T // TC,),
        in_specs=[
            pl.BlockSpec((B, TC, D), lambda i: (0, i, 0)),
            pl.BlockSpec((B, H), lambda i: (0, 0)),
            pl.BlockSpec((D, H4), lambda i: (0, 0)),
            pl.BlockSpec((H, H4), lambda i: (0, 0)),
            pl.BlockSpec((1, H4), lambda i: (0, 0)),
        ],
        out_specs=[
            pl.BlockSpec((B, TC, H), lambda i: (0, i, 0)),
            pl.BlockSpec((1, B, H), lambda i: (0, 0, 0)),
            pl.BlockSpec((1, B, H), lambda i: (0, 0, 0)),
        ],
        out_shape=[
            jax.ShapeDtypeStruct((B, T, H), jnp.float32),
            jax.ShapeDtypeStruct((1, B, H), jnp.float32),
            jax.ShapeDtypeStruct((1, B, H), jnp.float32),
        ],
        scratch_shapes=[
            pltpu.VMEM((B, TC, H4), jnp.float32),
            pltpu.VMEM((B, H), jnp.float32),
            pltpu.VMEM((B, H), jnp.float32),
        ],
    )(input, lens, wih_t, whh_t, bias)

    return (out_tm, h_n, c_n)


# speedup vs baseline: 1.0244x; 1.0109x over previous
import functools

import jax
import jax.numpy as jnp
from jax.experimental import pallas as pl
from jax.experimental.pallas import tpu as pltpu


def _lstm_body(x_ref, lens_ref, wih_ref, whh_ref, b_ref,
               out_ref, hn_ref, cn_ref,
               g_s, h_s, c_s, *, tc, b, h):
    i = pl.program_id(0)
    nblk = pl.num_programs(0)

    @pl.when(i == 0)
    def _init():
        h_s[...] = jnp.zeros_like(h_s)
        c_s[...] = jnp.zeros_like(c_s)

    xm = x_ref[...].reshape(b * tc, x_ref.shape[-1])
    g_s[...] = (jnp.dot(xm, wih_ref[...], preferred_element_type=jnp.float32)
                + b_ref[...]).reshape(b, tc, 4 * h)

    lens_v = lens_ref[...]

    def step(t, carry):
        hc, cc = carry
        hb = hc.astype(jnp.bfloat16)
        gt = g_s[:, t, :]
        ig = jax.nn.sigmoid(gt[:, :h]
                            + jnp.dot(hb, whh_ref[:, :h],
                                      preferred_element_type=jnp.float32))
        gg = jnp.tanh(gt[:, 2 * h:3 * h]
                      + jnp.dot(hb, whh_ref[:, 2 * h:3 * h],
                                preferred_element_type=jnp.float32))
        fg = jax.nn.sigmoid(gt[:, h:2 * h]
                            + jnp.dot(hb, whh_ref[:, h:2 * h],
                                      preferred_element_type=jnp.float32))
        og = jax.nn.sigmoid(gt[:, 3 * h:]
                            + jnp.dot(hb, whh_ref[:, 3 * h:],
                                      preferred_element_type=jnp.float32))
        c_new = fg * cc + ig * gg
        h_new = og * jnp.tanh(c_new)
        tg = i * tc + t
        m = (tg < lens_v).astype(jnp.float32)
        out_ref[:, t, :] = m * h_new
        return (hc + m * (h_new - hc), cc + m * (c_new - cc))

    hf, cf = jax.lax.fori_loop(0, tc, step, (h_s[...], c_s[...]), unroll=16)
    h_s[...] = hf
    c_s[...] = cf

    @pl.when(i == nblk - 1)
    def _fin():
        hn_ref[...] = hf[None]
        cn_ref[...] = cf[None]


def kernel(input, seq_lengths, W_ih, W_hh, b_ih, b_hh):
    B, T, D = input.shape
    H = W_hh.shape[1]
    H4 = 4 * H
    TC = 128

    wih_t = jnp.transpose(W_ih)
    whh_t = jnp.transpose(W_hh).astype(jnp.bfloat16)
    bias = (b_ih + b_hh)[None, :]
    lens = jnp.broadcast_to(
        seq_lengths.astype(jnp.int32)[:, None], (B, H))

    body = functools.partial(_lstm_body, tc=TC, b=B, h=H)
    out_tm, h_n, c_n = pl.pallas_call(
        body,
        grid=(T // TC,),
        in_specs=[
            pl.BlockSpec((B, TC, D), lambda i: (0, i, 0)),
            pl.BlockSpec((B, H), lambda i: (0, 0)),
            pl.BlockSpec((D, H4), lambda i: (0, 0)),
            pl.BlockSpec((H, H4), lambda i: (0, 0)),
            pl.BlockSpec((1, H4), lambda i: (0, 0)),
        ],
        out_specs=[
            pl.BlockSpec((B, TC, H), lambda i: (0, i, 0)),
            pl.BlockSpec((1, B, H), lambda i: (0, 0, 0)),
            pl.BlockSpec((1, B, H), lambda i: (0, 0, 0)),
        ],
        out_shape=[
            jax.ShapeDtypeStruct((B, T, H), jnp.float32),
            jax.ShapeDtypeStruct((1, B, H), jnp.float32),
            jax.ShapeDtypeStruct((1, B, H), jnp.float32),
        ],
        scratch_shapes=[
            pltpu.VMEM((B, TC, H4), jnp.float32),
            pltpu.VMEM((B, H), jnp.float32),
            pltpu.VMEM((B, H), jnp.float32),
        ],
    )(input, lens, wih_t, whh_t, bias)

    return (out_tm, h_n, c_n)


# speedup vs baseline: 1.0294x; 1.0049x over previous
import functools

import jax
import jax.numpy as jnp
from jax.experimental import pallas as pl
from jax.experimental.pallas import tpu as pltpu


def _lstm_body(x_ref, lens_ref, wih_ref, whh_ref, b_ref,
               out_ref, hn_ref, cn_ref,
               g_s, h_s, c_s, *, tc, b, h):
    i = pl.program_id(0)
    nblk = pl.num_programs(0)

    @pl.when(i == 0)
    def _init():
        h_s[...] = jnp.zeros_like(h_s)
        c_s[...] = jnp.zeros_like(c_s)

    xm = x_ref[...].reshape(b * tc, x_ref.shape[-1])
    g_s[...] = (jnp.dot(xm, wih_ref[...], preferred_element_type=jnp.float32)
                + b_ref[...]).reshape(b, tc, 4 * h)

    lens_v = lens_ref[...]

    def step(t, carry):
        hc, cc = carry
        hb = hc.astype(jnp.bfloat16)
        gt = g_s[:, t, :]
        ig = jax.nn.sigmoid(gt[:, :h]
                            + jnp.dot(hb, whh_ref[:, :h],
                                      preferred_element_type=jnp.float32))
        gg = jnp.tanh(gt[:, 2 * h:3 * h]
                      + jnp.dot(hb, whh_ref[:, 2 * h:3 * h],
                                preferred_element_type=jnp.float32))
        fg = jax.nn.sigmoid(gt[:, h:2 * h]
                            + jnp.dot(hb, whh_ref[:, h:2 * h],
                                      preferred_element_type=jnp.float32))
        og = jax.nn.sigmoid(gt[:, 3 * h:]
                            + jnp.dot(hb, whh_ref[:, 3 * h:],
                                      preferred_element_type=jnp.float32))
        c_new = fg * cc + ig * gg
        h_new = og * jnp.tanh(c_new)
        tg = i * tc + t
        m = (tg < lens_v).astype(jnp.float32)
        out_ref[:, t, :] = m * h_new
        return (hc + m * (h_new - hc), cc + m * (c_new - cc))

    hf, cf = jax.lax.fori_loop(0, tc, step, (h_s[...], c_s[...]), unroll=32)
    h_s[...] = hf
    c_s[...] = cf

    @pl.when(i == nblk - 1)
    def _fin():
        hn_ref[...] = hf[None]
        cn_ref[...] = cf[None]


def kernel(input, seq_lengths, W_ih, W_hh, b_ih, b_hh):
    B, T, D = input.shape
    H = W_hh.shape[1]
    H4 = 4 * H
    TC = 128

    wih_t = jnp.transpose(W_ih)
    whh_t = jnp.transpose(W_hh).astype(jnp.bfloat16)
    bias = (b_ih + b_hh)[None, :]
    lens = jnp.broadcast_to(
        seq_lengths.astype(jnp.int32)[:, None], (B, H))

    body = functools.partial(_lstm_body, tc=TC, b=B, h=H)
    out_tm, h_n, c_n = pl.pallas_call(
        body,
        grid=(T // TC,),
        in_specs=[
            pl.BlockSpec((B, TC, D), lambda i: (0, i, 0)),
            pl.BlockSpec((B, H), lambda i: (0, 0)),
            pl.BlockSpec((D, H4), lambda i: (0, 0)),
            pl.BlockSpec((H, H4), lambda i: (0, 0)),
            pl.BlockSpec((1, H4), lambda i: (0, 0)),
        ],
        out_specs=[
            pl.BlockSpec((B, TC, H), lambda i: (0, i, 0)),
            pl.BlockSpec((1, B, H), lambda i: (0, 0, 0)),
            pl.BlockSpec((1, B, H), lambda i: (0, 0, 0)),
        ],
        out_shape=[
            jax.ShapeDtypeStruct((B, T, H), jnp.float32),
            jax.ShapeDtypeStruct((1, B, H), jnp.float32),
            jax.ShapeDtypeStruct((1, B, H), jnp.float32),
        ],
        scratch_shapes=[
            pltpu.VMEM((B, TC, H4), jnp.float32),
            pltpu.VMEM((B, H), jnp.float32),
            pltpu.VMEM((B, H), jnp.float32),
        ],
    )(input, lens, wih_t, whh_t, bias)

    return (out_tm, h_n, c_n)
